# baseline (speedup 1.0000x reference)
import functools

import jax
import jax.numpy as jnp
from jax import lax
from jax.experimental import pallas as pl
from jax.experimental.pallas import tpu as pltpu
from jax.experimental.pallas import tpu_sc as plsc

N = 10000
E = 320000
DF = 128
DH = 128
DO = 64

NC = 2
NS = 16
NW = NC * NS
L = 16

K = 128
TCH = 2560
EPAD = TCH * K
LAG = 8

DEG_SPLIT = (96, 64)
AGG128_SPLIT = None
AGG64_SPLIT = (108, 52)
NPAD = 10240
RPT = NPAD // NS
ZR = 64
DDEG = 16

BLK = 256
GRID = NPAD // BLK

_mesh = plsc.VectorSubcoreMesh(core_axis_name="c", subcore_axis_name="s")


def _zero_fill(ref, nrows, width):
    z = jnp.zeros((L,), jnp.float32)
    per_row = width // L

    def body(t, _):
        ref[t // per_row, pl.ds((t % per_row) * L, L)] = z
        return 0

    lax.fori_loop(0, nrows * per_row, body, 0)


def _deg_body(srcp2, dstp2, out_o, out_i, sidx_all, didx_all, ones, zbuf,
              acc_o, acc_i, sem_o, sem_i):
    cid = lax.axis_index("c")
    sid = lax.axis_index("s")
    c0, c1 = DEG_SPLIT

    one = jnp.full((L,), 1.0, jnp.float32)

    def fill_ones(t, _):
        ones[t, pl.ds(0, L)] = one
        return 0

    lax.fori_loop(0, K, fill_ones, 0)
    _zero_fill(zbuf, ZR, DDEG)
    for t in range(RPT // ZR):
        r0 = sid * RPT + t * ZR
        pltpu.sync_copy(zbuf, acc_o.at[pl.ds(r0, ZR)])
        pltpu.sync_copy(zbuf, acc_i.at[pl.ds(r0, ZR)])
    plsc.subcore_barrier()

    def issue(j):
        pltpu.async_copy(ones, acc_o.at[sidx_all.at[j]], sem_o, add=True)
        pltpu.async_copy(ones, acc_i.at[didx_all.at[j]], sem_i, add=True)

    def drain(j):
        pltpu.make_async_copy(ones, acc_o.at[sidx_all.at[j]], sem_o).wait()
        pltpu.make_async_copy(ones, acc_i.at[didx_all.at[j]], sem_i).wait()

    def dpipe(base, n):
        pltpu.sync_copy(srcp2.at[pl.ds(base, n)], sidx_all.at[pl.ds(0, n)])
        pltpu.sync_copy(dstp2.at[pl.ds(base, n)], didx_all.at[pl.ds(0, n)])

        @pl.loop(0, LAG)
        def _prime(j):
            issue(j)

        @pl.loop(LAG, n)
        def _steady(j):
            issue(j)
            drain(j - LAG)

        @pl.loop(n - LAG, n)
        def _tail(j):
            drain(j)

    @pl.when(cid == 0)
    def _core0():
        dpipe(sid * c0, c0)

    @pl.when(cid != 0)
    def _core1():
        dpipe(NS * c0 + sid * c1, c1)

    plsc.subcore_barrier()
    r0 = sid * RPT
    pltpu.sync_copy(acc_o.at[pl.ds(r0, RPT)], out_o.at[cid, pl.ds(r0, RPT)])
    pltpu.sync_copy(acc_i.at[pl.ds(r0, RPT)], out_i.at[cid, pl.ds(r0, RPT)])


_sc_params = pltpu.CompilerParams(use_tc_tiling_on_sc=False)

_deg_kernel = pl.kernel(
    _deg_body,
    out_type=(jax.ShapeDtypeStruct((NC, NPAD, DDEG), jnp.float32),
              jax.ShapeDtypeStruct((NC, NPAD, DDEG), jnp.float32)),
    mesh=_mesh,
    compiler_params=_sc_params,
    scratch_types=[
        pltpu.VMEM((max(DEG_SPLIT), K), jnp.int32),
        pltpu.VMEM((max(DEG_SPLIT), K), jnp.int32),
        pltpu.VMEM((K, DDEG), jnp.float32),
        pltpu.VMEM((ZR, DDEG), jnp.float32),
        pltpu.VMEM_SHARED((NPAD, DDEG), jnp.float32),
        pltpu.VMEM_SHARED((NPAD, DDEG), jnp.float32),
        pltpu.SemaphoreType.DMA,
        pltpu.SemaphoreType.DMA,
    ],
)


def _make_agg(D, split):
    ncores = NC if split is not None else 1

    def body(table, srcp2, dstp2, out, sidx4, didx4, rows0, rows1,
             acc, isem, gsem, ssem):
        cid = lax.axis_index("c")
        sid = lax.axis_index("s")
        rows = (rows0, rows1)

        def i_issue(g, q):
            pltpu.async_copy(srcp2.at[g], sidx4.at[q], isem.at[q])
            pltpu.async_copy(dstp2.at[g], didx4.at[q], isem.at[q])

        def i_wait(g, q):
            pltpu.make_async_copy(srcp2.at[g], sidx4.at[q],
                                  isem.at[q]).wait()
            pltpu.make_async_copy(dstp2.at[g], didx4.at[q],
                                  isem.at[q]).wait()

        def g_issue(b, q):
            pltpu.async_copy(table.at[sidx4.at[q]], rows[b], gsem.at[b])

        def g_wait(b, q):
            pltpu.make_async_copy(table.at[sidx4.at[q]], rows[b],
                                  gsem.at[b]).wait()

        def s_issue(b, q):
            pltpu.async_copy(rows[b], acc.at[didx4.at[q]], ssem.at[b],
                             add=True)

        def s_wait(b, q):
            pltpu.make_async_copy(rows[b], acc.at[didx4.at[q]],
                                  ssem.at[b]).wait()

        _zero_fill(rows0, K, D)
        for t in range(RPT // K):
            pltpu.sync_copy(rows0, acc.at[pl.ds(sid * RPT + t * K, K)])
        plsc.subcore_barrier()

        def pipe(base, n):
            for q in range(4):
                i_issue(base + q, q)
            i_wait(base, 0)
            g_issue(0, 0)
            g_wait(0, 0)
            s_issue(0, 0)
            i_wait(base + 1, 1)
            g_issue(1, 1)

            @pl.loop(0, (n - 4) // 4)
            def _steady(p):
                for r in range(4):
                    j = 4 * p + 1 + r
                    b = (1 + r) & 1
                    g_wait(b, (1 + r) & 3)
                    s_issue(b, (1 + r) & 3)
                    s_wait(b ^ 1, r & 3)
                    i_issue(base + j + 3, r & 3)
                    i_wait(base + j + 1, (2 + r) & 3)
                    g_issue(b ^ 1, (2 + r) & 3)

            for j in (n - 3, n - 2):
                b = j & 1
                g_wait(b, j & 3)
                s_issue(b, j & 3)
                s_wait(b ^ 1, (j - 1) & 3)
                i_wait(base + j + 1, (j + 1) & 3)
                g_issue(b ^ 1, (j + 1) & 3)
            jl = n - 1
            g_wait(jl & 1, jl & 3)
            s_issue(jl & 1, jl & 3)
            s_wait((jl - 1) & 1, (jl - 1) & 3)
            s_wait(jl & 1, jl & 3)

        if split is None:
            pipe(sid * (TCH // NS), TCH // NS)
        else:
            c0, c1 = split

            @pl.when(cid == 0)
            def _core0():
                pipe(sid * c0, c0)

            @pl.when(cid != 0)
            def _core1():
                pipe(NS * c0 + sid * c1, c1)

        plsc.subcore_barrier()
        r0 = sid * RPT
        pltpu.sync_copy(acc.at[pl.ds(r0, RPT)], out.at[cid, pl.ds(r0, RPT)])

    return pl.kernel(
        body,
        out_type=jax.ShapeDtypeStruct((ncores, NPAD, D), jnp.float32),
        mesh=plsc.VectorSubcoreMesh(core_axis_name="c", subcore_axis_name="s",
                                    num_cores=ncores),
        compiler_params=_sc_params,
        scratch_types=[
            pltpu.VMEM((4, K), jnp.int32),
            pltpu.VMEM((4, K), jnp.int32),
            pltpu.VMEM((K, D), jnp.float32),
            pltpu.VMEM((K, D), jnp.float32),
            pltpu.VMEM_SHARED((NPAD, D), jnp.float32),
            pltpu.SemaphoreType.DMA((4,)),
            pltpu.SemaphoreType.DMA((2,)),
            pltpu.SemaphoreType.DMA((2,)),
        ],
    )


_agg128 = _make_agg(DF, AGG128_SPLIT)
_agg64 = _make_agg(DO, AGG64_SPLIT)


def _prescale_body(emb_ref, dego_ref, degi_ref, t1_ref, rso_ref, rsi_ref):
    dego = dego_ref[0] + dego_ref[1]
    degi = degi_ref[0] + degi_ref[1]
    rso = lax.rsqrt(jnp.maximum(dego, 1.0))
    rsi = lax.rsqrt(jnp.maximum(degi, 1.0))
    rso_ref[...] = rso
    rsi_ref[...] = rsi
    t1_ref[...] = emb_ref[...] * rso[:, :1]


def _prescale(embp, deg_o, deg_i):
    return pl.pallas_call(
        _prescale_body,
        grid=(GRID,),
        in_specs=[
            pl.BlockSpec((BLK, DF), lambda i: (i, 0)),
            pl.BlockSpec((NC, BLK, DDEG), lambda i: (0, i, 0)),
            pl.BlockSpec((NC, BLK, DDEG), lambda i: (0, i, 0)),
        ],
        out_specs=[
            pl.BlockSpec((BLK, DF), lambda i: (i, 0)),
            pl.BlockSpec((BLK, DDEG), lambda i: (i, 0)),
            pl.BlockSpec((BLK, DDEG), lambda i: (i, 0)),
        ],
        out_shape=[
            jax.ShapeDtypeStruct((NPAD, DF), jnp.float32),
            jax.ShapeDtypeStruct((NPAD, DDEG), jnp.float32),
            jax.ShapeDtypeStruct((NPAD, DDEG), jnp.float32),
        ],
    )(embp, deg_o, deg_i)


def _mid_body(aggp_ref, rsi_ref, rso_ref, w1_ref, b1_ref, w2_ref, t2_ref):
    agg = aggp_ref[0]
    if aggp_ref.shape[0] == 2:
        agg = agg + aggp_ref[1]
    h = agg * rsi_ref[:, :1]
    h = jnp.dot(h, w1_ref[...], preferred_element_type=jnp.float32) + b1_ref[...]
    h = jnp.maximum(h, 0.0)
    h = h * rso_ref[:, :1]
    t2_ref[...] = jnp.dot(h, w2_ref[...], preferred_element_type=jnp.float32)


def _mid(aggp1, rsi, rso, W1, b1, W2):
    return pl.pallas_call(
        _mid_body,
        grid=(GRID,),
        in_specs=[
            pl.BlockSpec((aggp1.shape[0], BLK, DF), lambda i: (0, i, 0)),
            pl.BlockSpec((BLK, DDEG), lambda i: (i, 0)),
            pl.BlockSpec((BLK, DDEG), lambda i: (i, 0)),
            pl.BlockSpec((DF, DH), lambda i: (0, 0)),
            pl.BlockSpec((1, DH), lambda i: (0, 0)),
            pl.BlockSpec((DH, DO), lambda i: (0, 0)),
        ],
        out_specs=pl.BlockSpec((BLK, DO), lambda i: (i, 0)),
        out_shape=jax.ShapeDtypeStruct((NPAD, DO), jnp.float32),
    )(aggp1, rsi, rso, W1, b1, W2)


def _final_body(aggp_ref, rsi_ref, b2_ref, wf_ref, bf_ref, wb_ref, bb_ref, out_ref):
    agg = aggp_ref[0]
    if aggp_ref.shape[0] == 2:
        agg = agg + aggp_ref[1]
    h2 = agg * rsi_ref[:, :1] + b2_ref[...]

    def lstm(w_ref, bias_ref):
        g = jnp.dot(h2, w_ref[...], preferred_element_type=jnp.float32) + bias_ref[...]
        gi = jax.nn.sigmoid(g[:, 0:32])
        gg = jnp.tanh(g[:, 64:96])
        go = jax.nn.sigmoid(g[:, 96:128])
        return go * jnp.tanh(gi * gg)

    out_ref[...] = jnp.concatenate([lstm(wf_ref, bf_ref), lstm(wb_ref, bb_ref)], axis=1)


def _final(aggp2, rsi, b2, wfT, bf, wbT, bb):
    return pl.pallas_call(
        _final_body,
        grid=(GRID,),
        in_specs=[
            pl.BlockSpec((aggp2.shape[0], BLK, DO), lambda i: (0, i, 0)),
            pl.BlockSpec((BLK, DDEG), lambda i: (i, 0)),
            pl.BlockSpec((1, DO), lambda i: (0, 0)),
            pl.BlockSpec((DO, 4 * 32), lambda i: (0, 0)),
            pl.BlockSpec((1, 4 * 32), lambda i: (0, 0)),
            pl.BlockSpec((DO, 4 * 32), lambda i: (0, 0)),
            pl.BlockSpec((1, 4 * 32), lambda i: (0, 0)),
        ],
        out_specs=pl.BlockSpec((BLK, DO), lambda i: (i, 0)),
        out_shape=jax.ShapeDtypeStruct((NPAD, DO), jnp.float32),
    )(aggp2, rsi, b2, wfT, bf, wbT, bb)


def kernel(node_ids, edge_index, emb, W1, b1, W2, b2,
           Wih_f, Whh_f, bih_f, bhh_f, Wih_b, Whh_b, bih_b, bhh_b):
    f32 = jnp.float32
    src = edge_index[0]
    dst = edge_index[1]
    pad = jnp.full((EPAD - E,), N, jnp.int32)
    srcp = jnp.concatenate([src, pad]).reshape(TCH, K)
    dstp = jnp.concatenate([dst, pad]).reshape(TCH, K)
    embp = jnp.zeros((NPAD, DF), f32).at[:N].set(emb)

    deg_o, deg_i = _deg_kernel(srcp, dstp)
    table1, rso, rsi = _prescale(embp, deg_o, deg_i)
    aggp1 = _agg128(table1, srcp, dstp)
    table2 = _mid(aggp1, rsi, rso, W1, b1.reshape(1, DH), W2)
    aggp2 = _agg64(table2, srcp, dstp)
    bf = (bih_f + bhh_f).reshape(1, 4 * 32)
    bb = (bih_b + bhh_b).reshape(1, 4 * 32)
    outp = _final(aggp2, rsi, b2.reshape(1, DO), Wih_f.T, bf, Wih_b.T, bb)
    return outp[:N]


# speedup vs baseline: 1.5043x; 1.5043x over previous
import functools

import jax
import jax.numpy as jnp
from jax import lax
from jax.experimental import pallas as pl
from jax.experimental.pallas import tpu as pltpu
from jax.experimental.pallas import tpu_sc as plsc

N = 10000
E = 320000
DF = 128
DH = 128
DO = 64

NC = 2
NS = 16
NW = NC * NS
L = 16

K = 128
TCH = 2560
EPAD = TCH * K
LAG = 8

DEG_SPLIT = (96, 64)
AGG128_SPLIT = (152, 8)
AGG64_SPLIT = (80, 80)
NPAD = 10240
RPT = NPAD // NS
ZR = 64
DDEG = 16

BLK = 256
GRID = NPAD // BLK

_mesh = plsc.VectorSubcoreMesh(core_axis_name="c", subcore_axis_name="s")


def _zero_fill(ref, nrows, width):
    z = jnp.zeros((L,), jnp.float32)
    per_row = width // L

    def body(t, _):
        ref[t // per_row, pl.ds((t % per_row) * L, L)] = z
        return 0

    lax.fori_loop(0, nrows * per_row, body, 0)


def _deg_body(srcp2, dstp2, out_o, out_i, sidx_all, didx_all, ones, zbuf,
              acc_o, acc_i, sem_o, sem_i):
    cid = lax.axis_index("c")
    sid = lax.axis_index("s")
    c0, c1 = DEG_SPLIT

    one = jnp.full((L,), 1.0, jnp.float32)

    def fill_ones(t, _):
        ones[t, pl.ds(0, L)] = one
        return 0

    lax.fori_loop(0, K, fill_ones, 0)
    _zero_fill(zbuf, ZR, DDEG)
    for t in range(RPT // ZR):
        r0 = sid * RPT + t * ZR
        pltpu.sync_copy(zbuf, acc_o.at[pl.ds(r0, ZR)])
        pltpu.sync_copy(zbuf, acc_i.at[pl.ds(r0, ZR)])
    plsc.subcore_barrier()

    def issue(j):
        pltpu.async_copy(ones, acc_o.at[sidx_all.at[j]], sem_o, add=True)
        pltpu.async_copy(ones, acc_i.at[didx_all.at[j]], sem_i, add=True)

    def drain(j):
        pltpu.make_async_copy(ones, acc_o.at[sidx_all.at[j]], sem_o).wait()
        pltpu.make_async_copy(ones, acc_i.at[didx_all.at[j]], sem_i).wait()

    def dpipe(base, n):
        pltpu.sync_copy(srcp2.at[pl.ds(base, n)], sidx_all.at[pl.ds(0, n)])
        pltpu.sync_copy(dstp2.at[pl.ds(base, n)], didx_all.at[pl.ds(0, n)])

        @pl.loop(0, LAG)
        def _prime(j):
            issue(j)

        @pl.loop(LAG, n)
        def _steady(j):
            issue(j)
            drain(j - LAG)

        @pl.loop(n - LAG, n)
        def _tail(j):
            drain(j)

    @pl.when(cid == 0)
    def _core0():
        dpipe(sid * c0, c0)

    @pl.when(cid != 0)
    def _core1():
        dpipe(NS * c0 + sid * c1, c1)

    plsc.subcore_barrier()
    r0 = sid * RPT
    pltpu.sync_copy(acc_o.at[pl.ds(r0, RPT)], out_o.at[cid, pl.ds(r0, RPT)])
    pltpu.sync_copy(acc_i.at[pl.ds(r0, RPT)], out_i.at[cid, pl.ds(r0, RPT)])


_sc_params = pltpu.CompilerParams(use_tc_tiling_on_sc=False)

_deg_kernel = pl.kernel(
    _deg_body,
    out_type=(jax.ShapeDtypeStruct((NC, NPAD, DDEG), jnp.float32),
              jax.ShapeDtypeStruct((NC, NPAD, DDEG), jnp.float32)),
    mesh=_mesh,
    compiler_params=_sc_params,
    scratch_types=[
        pltpu.VMEM((max(DEG_SPLIT), K), jnp.int32),
        pltpu.VMEM((max(DEG_SPLIT), K), jnp.int32),
        pltpu.VMEM((K, DDEG), jnp.float32),
        pltpu.VMEM((ZR, DDEG), jnp.float32),
        pltpu.VMEM_SHARED((NPAD, DDEG), jnp.float32),
        pltpu.VMEM_SHARED((NPAD, DDEG), jnp.float32),
        pltpu.SemaphoreType.DMA,
        pltpu.SemaphoreType.DMA,
    ],
)


def _make_agg(D, split, spmem_table=False):
    ncores = NC if split is not None else 1

    def body(table_hbm, srcp2, dstp2, out, *scratch):
        if spmem_table:
            sidx4, didx4, rows0, rows1, acc, table_s, isem, gsem, ssem = scratch
            table = table_s
        else:
            sidx4, didx4, rows0, rows1, acc, isem, gsem, ssem = scratch
            table = table_hbm
        cid = lax.axis_index("c")
        sid = lax.axis_index("s")
        rows = (rows0, rows1)

        def i_issue(g, q):
            pltpu.async_copy(srcp2.at[g], sidx4.at[q], isem.at[q])
            pltpu.async_copy(dstp2.at[g], didx4.at[q], isem.at[q])

        def i_wait(g, q):
            pltpu.make_async_copy(srcp2.at[g], sidx4.at[q],
                                  isem.at[q]).wait()
            pltpu.make_async_copy(dstp2.at[g], didx4.at[q],
                                  isem.at[q]).wait()

        def g_issue(b, q):
            pltpu.async_copy(table.at[sidx4.at[q]], rows[b], gsem.at[b])

        def g_wait(b, q):
            pltpu.make_async_copy(table.at[sidx4.at[q]], rows[b],
                                  gsem.at[b]).wait()

        def s_issue(b, q):
            pltpu.async_copy(rows[b], acc.at[didx4.at[q]], ssem.at[b],
                             add=True)

        def s_wait(b, q):
            pltpu.make_async_copy(rows[b], acc.at[didx4.at[q]],
                                  ssem.at[b]).wait()

        _zero_fill(rows0, K, D)
        for t in range(RPT // K):
            pltpu.sync_copy(rows0, acc.at[pl.ds(sid * RPT + t * K, K)])
        if spmem_table:
            r0 = sid * RPT
            pltpu.sync_copy(table_hbm.at[pl.ds(r0, RPT)],
                            table_s.at[pl.ds(r0, RPT)])
        plsc.subcore_barrier()

        def pipe(base, n):
            for q in range(4):
                i_issue(base + q, q)
            i_wait(base, 0)
            g_issue(0, 0)
            g_wait(0, 0)
            s_issue(0, 0)
            i_wait(base + 1, 1)
            g_issue(1, 1)

            @pl.loop(0, (n - 4) // 4)
            def _steady(p):
                for r in range(4):
                    j = 4 * p + 1 + r
                    b = (1 + r) & 1
                    g_wait(b, (1 + r) & 3)
                    s_issue(b, (1 + r) & 3)
                    s_wait(b ^ 1, r & 3)
                    i_issue(base + j + 3, r & 3)
                    i_wait(base + j + 1, (2 + r) & 3)
                    g_issue(b ^ 1, (2 + r) & 3)

            for j in (n - 3, n - 2):
                b = j & 1
                g_wait(b, j & 3)
                s_issue(b, j & 3)
                s_wait(b ^ 1, (j - 1) & 3)
                i_wait(base + j + 1, (j + 1) & 3)
                g_issue(b ^ 1, (j + 1) & 3)
            jl = n - 1
            g_wait(jl & 1, jl & 3)
            s_issue(jl & 1, jl & 3)
            s_wait((jl - 1) & 1, (jl - 1) & 3)
            s_wait(jl & 1, jl & 3)

        if split is None:
            pipe(sid * (TCH // NS), TCH // NS)
        else:
            c0, c1 = split

            @pl.when(cid == 0)
            def _core0():
                pipe(sid * c0, c0)

            @pl.when(cid != 0)
            def _core1():
                pipe(NS * c0 + sid * c1, c1)

        plsc.subcore_barrier()
        r0 = sid * RPT
        pltpu.sync_copy(acc.at[pl.ds(r0, RPT)], out.at[cid, pl.ds(r0, RPT)])

    return pl.kernel(
        body,
        out_type=jax.ShapeDtypeStruct((ncores, NPAD, D), jnp.float32),
        mesh=plsc.VectorSubcoreMesh(core_axis_name="c", subcore_axis_name="s",
                                    num_cores=ncores),
        compiler_params=_sc_params,
        scratch_types=(
            [pltpu.VMEM((4, K), jnp.int32),
             pltpu.VMEM((4, K), jnp.int32),
             pltpu.VMEM((K, D), jnp.float32),
             pltpu.VMEM((K, D), jnp.float32),
             pltpu.VMEM_SHARED((NPAD, D), jnp.float32)]
            + ([pltpu.VMEM_SHARED((NPAD, D), jnp.float32)] if spmem_table
               else [])
            + [pltpu.SemaphoreType.DMA((4,)),
               pltpu.SemaphoreType.DMA((2,)),
               pltpu.SemaphoreType.DMA((2,))]
        ),
    )


_agg128 = _make_agg(DF, AGG128_SPLIT)
_agg64 = _make_agg(DO, AGG64_SPLIT, spmem_table=True)


def _prescale_body(emb_ref, dego_ref, degi_ref, t1_ref, rso_ref, rsi_ref):
    dego = dego_ref[0] + dego_ref[1]
    degi = degi_ref[0] + degi_ref[1]
    rso = lax.rsqrt(jnp.maximum(dego, 1.0))
    rsi = lax.rsqrt(jnp.maximum(degi, 1.0))
    rso_ref[...] = rso
    rsi_ref[...] = rsi
    t1_ref[...] = emb_ref[...] * rso[:, :1]


def _prescale(embp, deg_o, deg_i):
    return pl.pallas_call(
        _prescale_body,
        grid=(GRID,),
        in_specs=[
            pl.BlockSpec((BLK, DF), lambda i: (i, 0)),
            pl.BlockSpec((NC, BLK, DDEG), lambda i: (0, i, 0)),
            pl.BlockSpec((NC, BLK, DDEG), lambda i: (0, i, 0)),
        ],
        out_specs=[
            pl.BlockSpec((BLK, DF), lambda i: (i, 0)),
            pl.BlockSpec((BLK, DDEG), lambda i: (i, 0)),
            pl.BlockSpec((BLK, DDEG), lambda i: (i, 0)),
        ],
        out_shape=[
            jax.ShapeDtypeStruct((NPAD, DF), jnp.float32),
            jax.ShapeDtypeStruct((NPAD, DDEG), jnp.float32),
            jax.ShapeDtypeStruct((NPAD, DDEG), jnp.float32),
        ],
    )(embp, deg_o, deg_i)


def _mid_body(aggp_ref, rsi_ref, rso_ref, w1_ref, b1_ref, w2_ref, t2_ref):
    agg = aggp_ref[0]
    if aggp_ref.shape[0] == 2:
        agg = agg + aggp_ref[1]
    h = agg * rsi_ref[:, :1]
    h = jnp.dot(h, w1_ref[...], preferred_element_type=jnp.float32) + b1_ref[...]
    h = jnp.maximum(h, 0.0)
    h = h * rso_ref[:, :1]
    t2_ref[...] = jnp.dot(h, w2_ref[...], preferred_element_type=jnp.float32)


def _mid(aggp1, rsi, rso, W1, b1, W2):
    return pl.pallas_call(
        _mid_body,
        grid=(GRID,),
        in_specs=[
            pl.BlockSpec((aggp1.shape[0], BLK, DF), lambda i: (0, i, 0)),
            pl.BlockSpec((BLK, DDEG), lambda i: (i, 0)),
            pl.BlockSpec((BLK, DDEG), lambda i: (i, 0)),
            pl.BlockSpec((DF, DH), lambda i: (0, 0)),
            pl.BlockSpec((1, DH), lambda i: (0, 0)),
            pl.BlockSpec((DH, DO), lambda i: (0, 0)),
        ],
        out_specs=pl.BlockSpec((BLK, DO), lambda i: (i, 0)),
        out_shape=jax.ShapeDtypeStruct((NPAD, DO), jnp.float32),
    )(aggp1, rsi, rso, W1, b1, W2)


def _final_body(aggp_ref, rsi_ref, b2_ref, wf_ref, bf_ref, wb_ref, bb_ref, out_ref):
    agg = aggp_ref[0]
    if aggp_ref.shape[0] == 2:
        agg = agg + aggp_ref[1]
    h2 = agg * rsi_ref[:, :1] + b2_ref[...]

    def lstm(w_ref, bias_ref):
        g = jnp.dot(h2, w_ref[...], preferred_element_type=jnp.float32) + bias_ref[...]
        gi = jax.nn.sigmoid(g[:, 0:32])
        gg = jnp.tanh(g[:, 64:96])
        go = jax.nn.sigmoid(g[:, 96:128])
        return go * jnp.tanh(gi * gg)

    out_ref[...] = jnp.concatenate([lstm(wf_ref, bf_ref), lstm(wb_ref, bb_ref)], axis=1)


def _final(aggp2, rsi, b2, wfT, bf, wbT, bb):
    return pl.pallas_call(
        _final_body,
        grid=(GRID,),
        in_specs=[
            pl.BlockSpec((aggp2.shape[0], BLK, DO), lambda i: (0, i, 0)),
            pl.BlockSpec((BLK, DDEG), lambda i: (i, 0)),
            pl.BlockSpec((1, DO), lambda i: (0, 0)),
            pl.BlockSpec((DO, 4 * 32), lambda i: (0, 0)),
            pl.BlockSpec((1, 4 * 32), lambda i: (0, 0)),
            pl.BlockSpec((DO, 4 * 32), lambda i: (0, 0)),
            pl.BlockSpec((1, 4 * 32), lambda i: (0, 0)),
        ],
        out_specs=pl.BlockSpec((BLK, DO), lambda i: (i, 0)),
        out_shape=jax.ShapeDtypeStruct((NPAD, DO), jnp.float32),
    )(aggp2, rsi, b2, wfT, bf, wbT, bb)


def kernel(node_ids, edge_index, emb, W1, b1, W2, b2,
           Wih_f, Whh_f, bih_f, bhh_f, Wih_b, Whh_b, bih_b, bhh_b):
    f32 = jnp.float32
    src = edge_index[0]
    dst = edge_index[1]
    pad = jnp.full((EPAD - E,), N, jnp.int32)
    srcp = jnp.concatenate([src, pad]).reshape(TCH, K)
    dstp = jnp.concatenate([dst, pad]).reshape(TCH, K)
    embp = jnp.zeros((NPAD, DF), f32).at[:N].set(emb)

    deg_o, deg_i = _deg_kernel(srcp, dstp)
    table1, rso, rsi = _prescale(embp, deg_o, deg_i)
    aggp1 = _agg128(table1, srcp, dstp)
    table2 = _mid(aggp1, rsi, rso, W1, b1.reshape(1, DH), W2)
    aggp2 = _agg64(table2, srcp, dstp)
    bf = (bih_f + bhh_f).reshape(1, 4 * 32)
    bb = (bih_b + bhh_b).reshape(1, 4 * 32)
    outp = _final(aggp2, rsi, b2.reshape(1, DO), Wih_f.T, bf, Wih_b.T, bb)
    return outp[:N]


# speedup vs baseline: 2.2842x; 1.5184x over previous
import functools

import jax
import jax.numpy as jnp
from jax import lax
from jax.experimental import pallas as pl
from jax.experimental.pallas import tpu as pltpu
from jax.experimental.pallas import tpu_sc as plsc

N = 10000
E = 320000
DF = 128
DH = 128
DO = 64

NC = 2
NS = 16
NW = NC * NS
L = 16

K = 128
TCH = 2560
EPAD = TCH * K
LAG = 8

DEG_SPLIT = (96, 64)
AGG64_SPLIT = (80, 80)
DCOL = DF // 2
NPAD = 10240
RPT = NPAD // NS
ZR = 64
DDEG = 16

BLK = 256
GRID = NPAD // BLK

_mesh = plsc.VectorSubcoreMesh(core_axis_name="c", subcore_axis_name="s")


def _zero_fill(ref, nrows, width):
    z = jnp.zeros((L,), jnp.float32)
    per_row = width // L

    def body(t, _):
        ref[t // per_row, pl.ds((t % per_row) * L, L)] = z
        return 0

    lax.fori_loop(0, nrows * per_row, body, 0)


def _deg_body(srcp2, dstp2, out_o, out_i, sidx_all, didx_all, ones, zbuf,
              acc_o, acc_i, sem_o, sem_i):
    cid = lax.axis_index("c")
    sid = lax.axis_index("s")
    c0, c1 = DEG_SPLIT

    one = jnp.full((L,), 1.0, jnp.float32)

    def fill_ones(t, _):
        ones[t, pl.ds(0, L)] = one
        return 0

    lax.fori_loop(0, K, fill_ones, 0)
    _zero_fill(zbuf, ZR, DDEG)
    for t in range(RPT // ZR):
        r0 = sid * RPT + t * ZR
        pltpu.sync_copy(zbuf, acc_o.at[pl.ds(r0, ZR)])
        pltpu.sync_copy(zbuf, acc_i.at[pl.ds(r0, ZR)])
    plsc.subcore_barrier()

    def issue(j):
        pltpu.async_copy(ones, acc_o.at[sidx_all.at[j]], sem_o, add=True)
        pltpu.async_copy(ones, acc_i.at[didx_all.at[j]], sem_i, add=True)

    def drain(j):
        pltpu.make_async_copy(ones, acc_o.at[sidx_all.at[j]], sem_o).wait()
        pltpu.make_async_copy(ones, acc_i.at[didx_all.at[j]], sem_i).wait()

    def dpipe(base, n):
        pltpu.sync_copy(srcp2.at[pl.ds(base, n)], sidx_all.at[pl.ds(0, n)])
        pltpu.sync_copy(dstp2.at[pl.ds(base, n)], didx_all.at[pl.ds(0, n)])

        @pl.loop(0, LAG)
        def _prime(j):
            issue(j)

        @pl.loop(LAG, n)
        def _steady(j):
            issue(j)
            drain(j - LAG)

        @pl.loop(n - LAG, n)
        def _tail(j):
            drain(j)

    @pl.when(cid == 0)
    def _core0():
        dpipe(sid * c0, c0)

    @pl.when(cid != 0)
    def _core1():
        dpipe(NS * c0 + sid * c1, c1)

    plsc.subcore_barrier()
    r0 = sid * RPT
    pltpu.sync_copy(acc_o.at[pl.ds(r0, RPT)], out_o.at[cid, pl.ds(r0, RPT)])
    pltpu.sync_copy(acc_i.at[pl.ds(r0, RPT)], out_i.at[cid, pl.ds(r0, RPT)])


_sc_params = pltpu.CompilerParams(use_tc_tiling_on_sc=False)

_deg_kernel = pl.kernel(
    _deg_body,
    out_type=(jax.ShapeDtypeStruct((NC, NPAD, DDEG), jnp.float32),
              jax.ShapeDtypeStruct((NC, NPAD, DDEG), jnp.float32)),
    mesh=_mesh,
    compiler_params=_sc_params,
    scratch_types=[
        pltpu.VMEM((max(DEG_SPLIT), K), jnp.int32),
        pltpu.VMEM((max(DEG_SPLIT), K), jnp.int32),
        pltpu.VMEM((K, DDEG), jnp.float32),
        pltpu.VMEM((ZR, DDEG), jnp.float32),
        pltpu.VMEM_SHARED((NPAD, DDEG), jnp.float32),
        pltpu.VMEM_SHARED((NPAD, DDEG), jnp.float32),
        pltpu.SemaphoreType.DMA,
        pltpu.SemaphoreType.DMA,
    ],
)


def _make_agg(D, split, spmem_table=False, col_split=False):
    ncores = 1 if split is None else NC

    def body(table_hbm, srcp2, dstp2, out, *scratch):
        if spmem_table:
            sidx4, didx4, rows0, rows1, acc, table_s, isem, gsem, ssem = scratch
            table = table_s
        else:
            sidx4, didx4, rows0, rows1, acc, isem, gsem, ssem = scratch
            table = table_hbm
        cid = lax.axis_index("c")
        sid = lax.axis_index("s")
        rows = (rows0, rows1)

        def i_issue(g, q):
            pltpu.async_copy(srcp2.at[g], sidx4.at[q], isem.at[q])
            pltpu.async_copy(dstp2.at[g], didx4.at[q], isem.at[q])

        def i_wait(g, q):
            pltpu.make_async_copy(srcp2.at[g], sidx4.at[q],
                                  isem.at[q]).wait()
            pltpu.make_async_copy(dstp2.at[g], didx4.at[q],
                                  isem.at[q]).wait()

        def g_issue(b, q):
            pltpu.async_copy(table.at[sidx4.at[q]], rows[b], gsem.at[b])

        def g_wait(b, q):
            pltpu.make_async_copy(table.at[sidx4.at[q]], rows[b],
                                  gsem.at[b]).wait()

        def s_issue(b, q):
            pltpu.async_copy(rows[b], acc.at[didx4.at[q]], ssem.at[b],
                             add=True)

        def s_wait(b, q):
            pltpu.make_async_copy(rows[b], acc.at[didx4.at[q]],
                                  ssem.at[b]).wait()

        _zero_fill(rows0, K, D)
        for t in range(RPT // K):
            pltpu.sync_copy(rows0, acc.at[pl.ds(sid * RPT + t * K, K)])
        if spmem_table:
            r0 = sid * RPT
            if col_split:
                pltpu.sync_copy(table_hbm.at[cid, pl.ds(r0, RPT)],
                                table_s.at[pl.ds(r0, RPT)])
            else:
                pltpu.sync_copy(table_hbm.at[pl.ds(r0, RPT)],
                                table_s.at[pl.ds(r0, RPT)])
        plsc.subcore_barrier()

        def pipe(base, n):
            for q in range(4):
                i_issue(base + q, q)
            i_wait(base, 0)
            g_issue(0, 0)
            g_wait(0, 0)
            s_issue(0, 0)
            i_wait(base + 1, 1)
            g_issue(1, 1)

            @pl.loop(0, (n - 4) // 4)
            def _steady(p):
                for r in range(4):
                    j = 4 * p + 1 + r
                    b = (1 + r) & 1
                    g_wait(b, (1 + r) & 3)
                    s_issue(b, (1 + r) & 3)
                    s_wait(b ^ 1, r & 3)
                    i_issue(base + j + 3, r & 3)
                    i_wait(base + j + 1, (2 + r) & 3)
                    g_issue(b ^ 1, (2 + r) & 3)

            for j in (n - 3, n - 2):
                b = j & 1
                g_wait(b, j & 3)
                s_issue(b, j & 3)
                s_wait(b ^ 1, (j - 1) & 3)
                i_wait(base + j + 1, (j + 1) & 3)
                g_issue(b ^ 1, (j + 1) & 3)
            jl = n - 1
            g_wait(jl & 1, jl & 3)
            s_issue(jl & 1, jl & 3)
            s_wait((jl - 1) & 1, (jl - 1) & 3)
            s_wait(jl & 1, jl & 3)

        if split is None or col_split:
            pipe(sid * (TCH // NS), TCH // NS)
        else:
            c0, c1 = split

            @pl.when(cid == 0)
            def _core0():
                pipe(sid * c0, c0)

            @pl.when(cid != 0)
            def _core1():
                pipe(NS * c0 + sid * c1, c1)

        plsc.subcore_barrier()
        r0 = sid * RPT
        pltpu.sync_copy(acc.at[pl.ds(r0, RPT)], out.at[cid, pl.ds(r0, RPT)])

    return pl.kernel(
        body,
        out_type=jax.ShapeDtypeStruct((ncores, NPAD, D), jnp.float32),
        mesh=plsc.VectorSubcoreMesh(core_axis_name="c", subcore_axis_name="s",
                                    num_cores=ncores),
        compiler_params=_sc_params,
        scratch_types=(
            [pltpu.VMEM((4, K), jnp.int32),
             pltpu.VMEM((4, K), jnp.int32),
             pltpu.VMEM((K, D), jnp.float32),
             pltpu.VMEM((K, D), jnp.float32),
             pltpu.VMEM_SHARED((NPAD, D), jnp.float32)]
            + ([pltpu.VMEM_SHARED((NPAD, D), jnp.float32)] if spmem_table
               else [])
            + [pltpu.SemaphoreType.DMA((4,)),
               pltpu.SemaphoreType.DMA((2,)),
               pltpu.SemaphoreType.DMA((2,))]
        ),
    )


_agg128 = _make_agg(DCOL, (0, 0), spmem_table=True, col_split=True)
_agg64 = _make_agg(DO, AGG64_SPLIT, spmem_table=True)


def _prescale_body(emb_ref, dego_ref, degi_ref, t1_ref, rso_ref, rsi_ref):
    dego = dego_ref[0] + dego_ref[1]
    degi = degi_ref[0] + degi_ref[1]
    rso = lax.rsqrt(jnp.maximum(dego, 1.0))
    rsi = lax.rsqrt(jnp.maximum(degi, 1.0))
    rso_ref[...] = rso
    rsi_ref[...] = rsi
    scaled = emb_ref[...] * rso[:, :1]
    t1_ref[0] = scaled[:, :DCOL]
    t1_ref[1] = scaled[:, DCOL:]


def _prescale(embp, deg_o, deg_i):
    return pl.pallas_call(
        _prescale_body,
        grid=(GRID,),
        in_specs=[
            pl.BlockSpec((BLK, DF), lambda i: (i, 0)),
            pl.BlockSpec((NC, BLK, DDEG), lambda i: (0, i, 0)),
            pl.BlockSpec((NC, BLK, DDEG), lambda i: (0, i, 0)),
        ],
        out_specs=[
            pl.BlockSpec((2, BLK, DCOL), lambda i: (0, i, 0)),
            pl.BlockSpec((BLK, DDEG), lambda i: (i, 0)),
            pl.BlockSpec((BLK, DDEG), lambda i: (i, 0)),
        ],
        out_shape=[
            jax.ShapeDtypeStruct((2, NPAD, DCOL), jnp.float32),
            jax.ShapeDtypeStruct((NPAD, DDEG), jnp.float32),
            jax.ShapeDtypeStruct((NPAD, DDEG), jnp.float32),
        ],
    )(embp, deg_o, deg_i)


def _mid_body(aggp_ref, rsi_ref, rso_ref, w1_ref, b1_ref, w2_ref, t2_ref):
    agg = jnp.concatenate([aggp_ref[0], aggp_ref[1]], axis=1)
    h = agg * rsi_ref[:, :1]
    h = jnp.dot(h, w1_ref[...], preferred_element_type=jnp.float32) + b1_ref[...]
    h = jnp.maximum(h, 0.0)
    h = h * rso_ref[:, :1]
    t2_ref[...] = jnp.dot(h, w2_ref[...], preferred_element_type=jnp.float32)


def _mid(aggp1, rsi, rso, W1, b1, W2):
    return pl.pallas_call(
        _mid_body,
        grid=(GRID,),
        in_specs=[
            pl.BlockSpec((2, BLK, DCOL), lambda i: (0, i, 0)),
            pl.BlockSpec((BLK, DDEG), lambda i: (i, 0)),
            pl.BlockSpec((BLK, DDEG), lambda i: (i, 0)),
            pl.BlockSpec((DF, DH), lambda i: (0, 0)),
            pl.BlockSpec((1, DH), lambda i: (0, 0)),
            pl.BlockSpec((DH, DO), lambda i: (0, 0)),
        ],
        out_specs=pl.BlockSpec((BLK, DO), lambda i: (i, 0)),
        out_shape=jax.ShapeDtypeStruct((NPAD, DO), jnp.float32),
    )(aggp1, rsi, rso, W1, b1, W2)


def _final_body(aggp_ref, rsi_ref, b2_ref, wf_ref, bf_ref, wb_ref, bb_ref, out_ref):
    agg = aggp_ref[0]
    if aggp_ref.shape[0] == 2:
        agg = agg + aggp_ref[1]
    h2 = agg * rsi_ref[:, :1] + b2_ref[...]

    def lstm(w_ref, bias_ref):
        g = jnp.dot(h2, w_ref[...], preferred_element_type=jnp.float32) + bias_ref[...]
        gi = jax.nn.sigmoid(g[:, 0:32])
        gg = jnp.tanh(g[:, 64:96])
        go = jax.nn.sigmoid(g[:, 96:128])
        return go * jnp.tanh(gi * gg)

    out_ref[...] = jnp.concatenate([lstm(wf_ref, bf_ref), lstm(wb_ref, bb_ref)], axis=1)


def _final(aggp2, rsi, b2, wfT, bf, wbT, bb):
    return pl.pallas_call(
        _final_body,
        grid=(GRID,),
        in_specs=[
            pl.BlockSpec((aggp2.shape[0], BLK, DO), lambda i: (0, i, 0)),
            pl.BlockSpec((BLK, DDEG), lambda i: (i, 0)),
            pl.BlockSpec((1, DO), lambda i: (0, 0)),
            pl.BlockSpec((DO, 4 * 32), lambda i: (0, 0)),
            pl.BlockSpec((1, 4 * 32), lambda i: (0, 0)),
            pl.BlockSpec((DO, 4 * 32), lambda i: (0, 0)),
            pl.BlockSpec((1, 4 * 32), lambda i: (0, 0)),
        ],
        out_specs=pl.BlockSpec((BLK, DO), lambda i: (i, 0)),
        out_shape=jax.ShapeDtypeStruct((NPAD, DO), jnp.float32),
    )(aggp2, rsi, b2, wfT, bf, wbT, bb)


def kernel(node_ids, edge_index, emb, W1, b1, W2, b2,
           Wih_f, Whh_f, bih_f, bhh_f, Wih_b, Whh_b, bih_b, bhh_b):
    f32 = jnp.float32
    src = edge_index[0]
    dst = edge_index[1]
    pad = jnp.full((EPAD - E,), N, jnp.int32)
    srcp = jnp.concatenate([src, pad]).reshape(TCH, K)
    dstp = jnp.concatenate([dst, pad]).reshape(TCH, K)
    embp = jnp.zeros((NPAD, DF), f32).at[:N].set(emb)

    deg_o, deg_i = _deg_kernel(srcp, dstp)
    table1, rso, rsi = _prescale(embp, deg_o, deg_i)
    aggp1 = _agg128(table1, srcp, dstp)
    table2 = _mid(aggp1, rsi, rso, W1, b1.reshape(1, DH), W2)
    aggp2 = _agg64(table2, srcp, dstp)
    bf = (bih_f + bhh_f).reshape(1, 4 * 32)
    bb = (bih_b + bhh_b).reshape(1, 4 * 32)
    outp = _final(aggp2, rsi, b2.reshape(1, DO), Wih_f.T, bf, Wih_b.T, bb)
    return outp[:N]


# speedup vs baseline: 2.9413x; 1.2877x over previous
import functools

import jax
import jax.numpy as jnp
from jax import lax
from jax.experimental import pallas as pl
from jax.experimental.pallas import tpu as pltpu
from jax.experimental.pallas import tpu_sc as plsc

N = 10000
E = 320000
DF = 128
DH = 128
DO = 64

NC = 2
NS = 16
NW = NC * NS
L = 16

K = 128
TCH = 2560
EPAD = TCH * K
LAG = 8

DEG_SPLIT = (96, 64)
AGG64_SPLIT = (80, 80)
DCOL = DF // 2
NPAD = 10240
RPT = NPAD // NS
ZR = 64
DDEG = 16

BLK = 1024
GRID = NPAD // BLK

_mesh = plsc.VectorSubcoreMesh(core_axis_name="c", subcore_axis_name="s")


def _zero_fill(ref, nrows, width):
    z = jnp.zeros((L,), jnp.float32)
    per_row = width // L

    def body(t, _):
        ref[t // per_row, pl.ds((t % per_row) * L, L)] = z
        return 0

    lax.fori_loop(0, nrows * per_row, body, 0)


def _deg_body(srcp2, dstp2, out, sidx_all, didx_all, ones, zbuf,
              acc_o, acc_i, sem_o, sem_i):
    cid = lax.axis_index("c")
    sid = lax.axis_index("s")
    c0, c1 = DEG_SPLIT

    one = jnp.full((L,), 1.0, jnp.float32)

    def fill_ones(t, _):
        ones[t, pl.ds(0, L)] = one
        return 0

    lax.fori_loop(0, K, fill_ones, 0)
    _zero_fill(zbuf, ZR, DDEG)
    for t in range(RPT // ZR):
        r0 = sid * RPT + t * ZR
        pltpu.sync_copy(zbuf, acc_o.at[pl.ds(r0, ZR)])
        pltpu.sync_copy(zbuf, acc_i.at[pl.ds(r0, ZR)])
    plsc.subcore_barrier()

    def issue(j):
        pltpu.async_copy(ones, acc_o.at[sidx_all.at[j]], sem_o, add=True)
        pltpu.async_copy(ones, acc_i.at[didx_all.at[j]], sem_i, add=True)

    def drain(j):
        pltpu.make_async_copy(ones, acc_o.at[sidx_all.at[j]], sem_o).wait()
        pltpu.make_async_copy(ones, acc_i.at[didx_all.at[j]], sem_i).wait()

    def dpipe(base, n):
        pltpu.sync_copy(srcp2.at[pl.ds(base, n)], sidx_all.at[pl.ds(0, n)])
        pltpu.sync_copy(dstp2.at[pl.ds(base, n)], didx_all.at[pl.ds(0, n)])

        @pl.loop(0, LAG)
        def _prime(j):
            issue(j)

        @pl.loop(LAG, n)
        def _steady(j):
            issue(j)
            drain(j - LAG)

        @pl.loop(n - LAG, n)
        def _tail(j):
            drain(j)

    @pl.when(cid == 0)
    def _core0():
        dpipe(sid * c0, c0)

    @pl.when(cid != 0)
    def _core1():
        dpipe(NS * c0 + sid * c1, c1)

    plsc.subcore_barrier()
    r0 = sid * RPT
    pltpu.sync_copy(acc_o.at[pl.ds(r0, RPT)],
                    out.at[pl.ds(r0, RPT), pl.ds(cid * DDEG, DDEG)])
    pltpu.sync_copy(acc_i.at[pl.ds(r0, RPT)],
                    out.at[pl.ds(r0, RPT), pl.ds(32 + cid * DDEG, DDEG)])


_sc_params = pltpu.CompilerParams(use_tc_tiling_on_sc=False)

_deg_kernel = pl.kernel(
    _deg_body,
    out_type=jax.ShapeDtypeStruct((NPAD, DF), jnp.float32),
    mesh=_mesh,
    compiler_params=_sc_params,
    scratch_types=[
        pltpu.VMEM((max(DEG_SPLIT), K), jnp.int32),
        pltpu.VMEM((max(DEG_SPLIT), K), jnp.int32),
        pltpu.VMEM((K, DDEG), jnp.float32),
        pltpu.VMEM((ZR, DDEG), jnp.float32),
        pltpu.VMEM_SHARED((NPAD, DDEG), jnp.float32),
        pltpu.VMEM_SHARED((NPAD, DDEG), jnp.float32),
        pltpu.SemaphoreType.DMA,
        pltpu.SemaphoreType.DMA,
    ],
)


def _make_agg(D, split, spmem_table=False, col_split=False):
    ncores = 1 if split is None else NC

    def body(table_hbm, srcp2, dstp2, out, *scratch):
        if spmem_table:
            sidx4, didx4, rows0, rows1, acc, table_s, isem, gsem, ssem = scratch
            table = table_s
        else:
            sidx4, didx4, rows0, rows1, acc, isem, gsem, ssem = scratch
            table = table_hbm
        cid = lax.axis_index("c")
        sid = lax.axis_index("s")
        rows = (rows0, rows1)

        def i_issue(g, q):
            pltpu.async_copy(srcp2.at[g], sidx4.at[q], isem.at[q])
            pltpu.async_copy(dstp2.at[g], didx4.at[q], isem.at[q])

        def i_wait(g, q):
            pltpu.make_async_copy(srcp2.at[g], sidx4.at[q],
                                  isem.at[q]).wait()
            pltpu.make_async_copy(dstp2.at[g], didx4.at[q],
                                  isem.at[q]).wait()

        def g_issue(b, q):
            pltpu.async_copy(table.at[sidx4.at[q]], rows[b], gsem.at[b])

        def g_wait(b, q):
            pltpu.make_async_copy(table.at[sidx4.at[q]], rows[b],
                                  gsem.at[b]).wait()

        def s_issue(b, q):
            pltpu.async_copy(rows[b], acc.at[didx4.at[q]], ssem.at[b],
                             add=True)

        def s_wait(b, q):
            pltpu.make_async_copy(rows[b], acc.at[didx4.at[q]],
                                  ssem.at[b]).wait()

        _zero_fill(rows0, K, D)
        for t in range(RPT // K):
            pltpu.sync_copy(rows0, acc.at[pl.ds(sid * RPT + t * K, K)])
        if spmem_table:
            r0 = sid * RPT
            tc0 = cid * D if col_split else 0
            pltpu.sync_copy(table_hbm.at[pl.ds(r0, RPT), pl.ds(tc0, D)],
                            table_s.at[pl.ds(r0, RPT)])
        plsc.subcore_barrier()

        def pipe(base, n):
            for q in range(4):
                i_issue(base + q, q)
            i_wait(base, 0)
            g_issue(0, 0)
            g_wait(0, 0)
            s_issue(0, 0)
            i_wait(base + 1, 1)
            g_issue(1, 1)

            @pl.loop(0, (n - 4) // 4)
            def _steady(p):
                for r in range(4):
                    j = 4 * p + 1 + r
                    b = (1 + r) & 1
                    g_wait(b, (1 + r) & 3)
                    s_issue(b, (1 + r) & 3)
                    s_wait(b ^ 1, r & 3)
                    i_issue(base + j + 3, r & 3)
                    i_wait(base + j + 1, (2 + r) & 3)
                    g_issue(b ^ 1, (2 + r) & 3)

            for j in (n - 3, n - 2):
                b = j & 1
                g_wait(b, j & 3)
                s_issue(b, j & 3)
                s_wait(b ^ 1, (j - 1) & 3)
                i_wait(base + j + 1, (j + 1) & 3)
                g_issue(b ^ 1, (j + 1) & 3)
            jl = n - 1
            g_wait(jl & 1, jl & 3)
            s_issue(jl & 1, jl & 3)
            s_wait((jl - 1) & 1, (jl - 1) & 3)
            s_wait(jl & 1, jl & 3)

        if split is None or col_split:
            pipe(sid * (TCH // NS), TCH // NS)
        else:
            c0, c1 = split

            @pl.when(cid == 0)
            def _core0():
                pipe(sid * c0, c0)

            @pl.when(cid != 0)
            def _core1():
                pipe(NS * c0 + sid * c1, c1)

        plsc.subcore_barrier()
        r0 = sid * RPT
        pltpu.sync_copy(acc.at[pl.ds(r0, RPT)],
                        out.at[pl.ds(r0, RPT), pl.ds(cid * D, D)])

    return pl.kernel(
        body,
        out_type=jax.ShapeDtypeStruct((NPAD, DF), jnp.float32),
        mesh=plsc.VectorSubcoreMesh(core_axis_name="c", subcore_axis_name="s",
                                    num_cores=ncores),
        compiler_params=_sc_params,
        scratch_types=(
            [pltpu.VMEM((4, K), jnp.int32),
             pltpu.VMEM((4, K), jnp.int32),
             pltpu.VMEM((K, D), jnp.float32),
             pltpu.VMEM((K, D), jnp.float32),
             pltpu.VMEM_SHARED((NPAD, D), jnp.float32)]
            + ([pltpu.VMEM_SHARED((NPAD, D), jnp.float32)] if spmem_table
               else [])
            + [pltpu.SemaphoreType.DMA((4,)),
               pltpu.SemaphoreType.DMA((2,)),
               pltpu.SemaphoreType.DMA((2,))]
        ),
    )


_agg128 = _make_agg(DCOL, (0, 0), spmem_table=True, col_split=True)
_agg64 = _make_agg(DO, AGG64_SPLIT, spmem_table=True)


def _prescale_body(emb_ref, deg_ref, t1_ref, rso_ref, rsi_ref):
    deg = deg_ref[...]
    dego = deg[:, 0:16] + deg[:, 16:32]
    degi = deg[:, 32:48] + deg[:, 48:64]
    rso = lax.rsqrt(jnp.maximum(dego, 1.0))
    rsi = lax.rsqrt(jnp.maximum(degi, 1.0))
    rso_ref[...] = rso
    rsi_ref[...] = rsi
    t1_ref[...] = emb_ref[...] * rso[:, :1]


def _prescale(embp, deg):
    return pl.pallas_call(
        _prescale_body,
        grid=(GRID,),
        in_specs=[
            pl.BlockSpec((BLK, DF), lambda i: (i, 0)),
            pl.BlockSpec((BLK, DF), lambda i: (i, 0)),
        ],
        out_specs=[
            pl.BlockSpec((BLK, DF), lambda i: (i, 0)),
            pl.BlockSpec((BLK, DDEG), lambda i: (i, 0)),
            pl.BlockSpec((BLK, DDEG), lambda i: (i, 0)),
        ],
        out_shape=[
            jax.ShapeDtypeStruct((NPAD, DF), jnp.float32),
            jax.ShapeDtypeStruct((NPAD, DDEG), jnp.float32),
            jax.ShapeDtypeStruct((NPAD, DDEG), jnp.float32),
        ],
    )(embp, deg)


def _mid_body(agg_ref, rsi_ref, rso_ref, w1_ref, b1_ref, w2_ref, t2_ref):
    h = agg_ref[...] * rsi_ref[:, :1]
    h = jnp.dot(h, w1_ref[...], preferred_element_type=jnp.float32) + b1_ref[...]
    h = jnp.maximum(h, 0.0)
    h = h * rso_ref[:, :1]
    t2_ref[:, :DO] = jnp.dot(h, w2_ref[...], preferred_element_type=jnp.float32)


def _mid(agg1, rsi, rso, W1, b1, W2):
    return pl.pallas_call(
        _mid_body,
        grid=(GRID,),
        in_specs=[
            pl.BlockSpec((BLK, DF), lambda i: (i, 0)),
            pl.BlockSpec((BLK, DDEG), lambda i: (i, 0)),
            pl.BlockSpec((BLK, DDEG), lambda i: (i, 0)),
            pl.BlockSpec((DF, DH), lambda i: (0, 0)),
            pl.BlockSpec((1, DH), lambda i: (0, 0)),
            pl.BlockSpec((DH, DO), lambda i: (0, 0)),
        ],
        out_specs=pl.BlockSpec((BLK, DF), lambda i: (i, 0)),
        out_shape=jax.ShapeDtypeStruct((NPAD, DF), jnp.float32),
    )(agg1, rsi, rso, W1, b1, W2)


def _final_body(aggp_ref, rsi_ref, b2_ref, wf_ref, bf_ref, wb_ref, bb_ref, out_ref):
    aggp = aggp_ref[...]
    agg = aggp[:, :DO] + aggp[:, DO:]
    h2 = agg * rsi_ref[:, :1] + b2_ref[...]

    def lstm(w_ref, bias_ref):
        g = jnp.dot(h2, w_ref[...], preferred_element_type=jnp.float32) + bias_ref[...]
        gi = jax.nn.sigmoid(g[:, 0:32])
        gg = jnp.tanh(g[:, 64:96])
        go = jax.nn.sigmoid(g[:, 96:128])
        return go * jnp.tanh(gi * gg)

    out_ref[...] = jnp.concatenate([lstm(wf_ref, bf_ref), lstm(wb_ref, bb_ref)], axis=1)


def _final(aggp2, rsi, b2, wfT, bf, wbT, bb):
    return pl.pallas_call(
        _final_body,
        grid=(GRID,),
        in_specs=[
            pl.BlockSpec((BLK, DF), lambda i: (i, 0)),
            pl.BlockSpec((BLK, DDEG), lambda i: (i, 0)),
            pl.BlockSpec((1, DO), lambda i: (0, 0)),
            pl.BlockSpec((DO, 4 * 32), lambda i: (0, 0)),
            pl.BlockSpec((1, 4 * 32), lambda i: (0, 0)),
            pl.BlockSpec((DO, 4 * 32), lambda i: (0, 0)),
            pl.BlockSpec((1, 4 * 32), lambda i: (0, 0)),
        ],
        out_specs=pl.BlockSpec((BLK, DO), lambda i: (i, 0)),
        out_shape=jax.ShapeDtypeStruct((NPAD, DO), jnp.float32),
    )(aggp2, rsi, b2, wfT, bf, wbT, bb)


def kernel(node_ids, edge_index, emb, W1, b1, W2, b2,
           Wih_f, Whh_f, bih_f, bhh_f, Wih_b, Whh_b, bih_b, bhh_b):
    f32 = jnp.float32
    src = edge_index[0]
    dst = edge_index[1]
    pad = jnp.full((EPAD - E,), N, jnp.int32)
    srcp = jnp.concatenate([src, pad]).reshape(TCH, K)
    dstp = jnp.concatenate([dst, pad]).reshape(TCH, K)
    embp = jnp.zeros((NPAD, DF), f32).at[:N].set(emb)

    deg = _deg_kernel(srcp, dstp)
    table1, rso, rsi = _prescale(embp, deg)
    aggp1 = _agg128(table1, srcp, dstp)
    table2 = _mid(aggp1, rsi, rso, W1, b1.reshape(1, DH), W2)
    aggp2 = _agg64(table2, srcp, dstp)
    bf = (bih_f + bhh_f).reshape(1, 4 * 32)
    bb = (bih_b + bhh_b).reshape(1, 4 * 32)
    outp = _final(aggp2, rsi, b2.reshape(1, DO), Wih_f.T, bf, Wih_b.T, bb)
    return outp[:N]


# speedup vs baseline: 3.0315x; 1.0307x over previous
import functools

import jax
import jax.numpy as jnp
from jax import lax
from jax.experimental import pallas as pl
from jax.experimental.pallas import tpu as pltpu
from jax.experimental.pallas import tpu_sc as plsc

N = 10000
E = 320000
DF = 128
DH = 128
DO = 64

NC = 2
NS = 16
NW = NC * NS
L = 16

K = 128
TCH = 2560
EPAD = TCH * K
LAG = 8

AGG64_SPLIT = (80, 80)
DCOL = DF // 2
NPAD = 10240
RPT = NPAD // NS
ZR = 64
DDEG = 8
TCH_RAW = E // K
DEG_C0, DEG_C1, DEG_EXTRA = 94, 62, 4

BLK = 1024
GRID = NPAD // BLK

_mesh = plsc.VectorSubcoreMesh(core_axis_name="c", subcore_axis_name="s")


def _zero_fill(ref, nrows, width):
    z = jnp.zeros((L,), jnp.float32)
    per_row = width // L

    def body(t, _):
        ref[t // per_row, pl.ds((t % per_row) * L, L)] = z
        return 0

    lax.fori_loop(0, nrows * per_row, body, 0)


def _deg_body(srcp2, dstp2, out, sidx_all, didx_all, ones, zbuf,
              acc_o, acc_i, sem_o, sem_i):
    cid = lax.axis_index("c")
    sid = lax.axis_index("s")

    one = jnp.full((L,), 1.0, jnp.float32)

    def fill_ones(t, _):
        ones[t, pl.ds(0, L)] = one
        return 0

    lax.fori_loop(0, K, fill_ones, 0)
    _zero_fill(zbuf, ZR, DDEG)
    for t in range(RPT // ZR):
        r0 = sid * RPT + t * ZR
        pltpu.sync_copy(zbuf, acc_o.at[pl.ds(r0, ZR)])
        pltpu.sync_copy(zbuf, acc_i.at[pl.ds(r0, ZR)])
    plsc.subcore_barrier()

    def issue(j):
        pltpu.async_copy(ones, acc_o.at[sidx_all.at[j]], sem_o, add=True)
        pltpu.async_copy(ones, acc_i.at[didx_all.at[j]], sem_i, add=True)

    def drain(j):
        pltpu.make_async_copy(ones, acc_o.at[sidx_all.at[j]], sem_o).wait()
        pltpu.make_async_copy(ones, acc_i.at[didx_all.at[j]], sem_i).wait()

    def dpipe(base, n):
        pltpu.sync_copy(srcp2.at[pl.ds(base, n)], sidx_all.at[pl.ds(0, n)])
        pltpu.sync_copy(dstp2.at[pl.ds(base, n)], didx_all.at[pl.ds(0, n)])

        @pl.loop(0, LAG)
        def _prime(j):
            issue(j)

        @pl.loop(LAG, n)
        def _steady(j):
            issue(j)
            drain(j - LAG)

        @pl.loop(n - LAG, n)
        def _tail(j):
            drain(j)

    @pl.when((cid == 0) & (sid == 0))
    def _tile00():
        dpipe(0, DEG_C0 + DEG_EXTRA)

    @pl.when((cid == 0) & (sid != 0))
    def _core0():
        dpipe(DEG_EXTRA + sid * DEG_C0, DEG_C0)

    @pl.when(cid != 0)
    def _core1():
        dpipe(NS * DEG_C0 + DEG_EXTRA + sid * DEG_C1, DEG_C1)

    plsc.subcore_barrier()
    r0 = sid * RPT
    pltpu.sync_copy(acc_o.at[pl.ds(r0, RPT)],
                    out.at[pl.ds(r0, RPT), pl.ds(cid * DDEG, DDEG)])
    pltpu.sync_copy(acc_i.at[pl.ds(r0, RPT)],
                    out.at[pl.ds(r0, RPT), pl.ds(16 + cid * DDEG, DDEG)])


_sc_params = pltpu.CompilerParams(use_tc_tiling_on_sc=False)

_deg_kernel = pl.kernel(
    _deg_body,
    out_type=jax.ShapeDtypeStruct((NPAD, DF), jnp.float32),
    mesh=_mesh,
    compiler_params=_sc_params,
    scratch_types=[
        pltpu.VMEM((DEG_C0 + DEG_EXTRA, K), jnp.int32),
        pltpu.VMEM((DEG_C0 + DEG_EXTRA, K), jnp.int32),
        pltpu.VMEM((K, DDEG), jnp.float32),
        pltpu.VMEM((ZR, DDEG), jnp.float32),
        pltpu.VMEM_SHARED((NPAD, DDEG), jnp.float32),
        pltpu.VMEM_SHARED((NPAD, DDEG), jnp.float32),
        pltpu.SemaphoreType.DMA,
        pltpu.SemaphoreType.DMA,
    ],
)


def _make_agg(D, split, spmem_table=False, col_split=False):
    ncores = 1 if split is None else NC

    def body(table_hbm, srcp2, dstp2, out, *scratch):
        if spmem_table:
            sidx4, didx4, rows0, rows1, acc, table_s, isem, gsem, ssem = scratch
            table = table_s
        else:
            sidx4, didx4, rows0, rows1, acc, isem, gsem, ssem = scratch
            table = table_hbm
        cid = lax.axis_index("c")
        sid = lax.axis_index("s")
        rows = (rows0, rows1)

        def i_issue(g, q):
            pltpu.async_copy(srcp2.at[g], sidx4.at[q], isem.at[q])
            pltpu.async_copy(dstp2.at[g], didx4.at[q], isem.at[q])

        def i_wait(g, q):
            pltpu.make_async_copy(srcp2.at[g], sidx4.at[q],
                                  isem.at[q]).wait()
            pltpu.make_async_copy(dstp2.at[g], didx4.at[q],
                                  isem.at[q]).wait()

        def g_issue(b, q):
            pltpu.async_copy(table.at[sidx4.at[q]], rows[b], gsem.at[b])

        def g_wait(b, q):
            pltpu.make_async_copy(table.at[sidx4.at[q]], rows[b],
                                  gsem.at[b]).wait()

        def s_issue(b, q):
            pltpu.async_copy(rows[b], acc.at[didx4.at[q]], ssem.at[b],
                             add=True)

        def s_wait(b, q):
            pltpu.make_async_copy(rows[b], acc.at[didx4.at[q]],
                                  ssem.at[b]).wait()

        _zero_fill(rows0, K, D)
        for t in range(RPT // K):
            pltpu.sync_copy(rows0, acc.at[pl.ds(sid * RPT + t * K, K)])
        if spmem_table:
            r0 = sid * RPT
            tc0 = cid * D if col_split else 0
            pltpu.sync_copy(table_hbm.at[pl.ds(r0, RPT), pl.ds(tc0, D)],
                            table_s.at[pl.ds(r0, RPT)])
        plsc.subcore_barrier()

        def pipe(base, n):
            for q in range(4):
                i_issue(base + q, q)
            i_wait(base, 0)
            g_issue(0, 0)
            g_wait(0, 0)
            s_issue(0, 0)
            i_wait(base + 1, 1)
            g_issue(1, 1)

            @pl.loop(0, (n - 4) // 4)
            def _steady(p):
                for r in range(4):
                    j = 4 * p + 1 + r
                    b = (1 + r) & 1
                    g_wait(b, (1 + r) & 3)
                    s_issue(b, (1 + r) & 3)
                    s_wait(b ^ 1, r & 3)
                    i_issue(base + j + 3, r & 3)
                    i_wait(base + j + 1, (2 + r) & 3)
                    g_issue(b ^ 1, (2 + r) & 3)

            for j in (n - 3, n - 2):
                b = j & 1
                g_wait(b, j & 3)
                s_issue(b, j & 3)
                s_wait(b ^ 1, (j - 1) & 3)
                i_wait(base + j + 1, (j + 1) & 3)
                g_issue(b ^ 1, (j + 1) & 3)
            jl = n - 1
            g_wait(jl & 1, jl & 3)
            s_issue(jl & 1, jl & 3)
            s_wait((jl - 1) & 1, (jl - 1) & 3)
            s_wait(jl & 1, jl & 3)

        if split is None or col_split:
            pipe(sid * (TCH // NS), TCH // NS)
        else:
            c0, c1 = split

            @pl.when(cid == 0)
            def _core0():
                pipe(sid * c0, c0)

            @pl.when(cid != 0)
            def _core1():
                pipe(NS * c0 + sid * c1, c1)

        plsc.subcore_barrier()
        r0 = sid * RPT
        pltpu.sync_copy(acc.at[pl.ds(r0, RPT)],
                        out.at[pl.ds(r0, RPT), pl.ds(cid * D, D)])

    return pl.kernel(
        body,
        out_type=jax.ShapeDtypeStruct((NPAD, DF), jnp.float32),
        mesh=plsc.VectorSubcoreMesh(core_axis_name="c", subcore_axis_name="s",
                                    num_cores=ncores),
        compiler_params=_sc_params,
        scratch_types=(
            [pltpu.VMEM((4, K), jnp.int32),
             pltpu.VMEM((4, K), jnp.int32),
             pltpu.VMEM((K, D), jnp.float32),
             pltpu.VMEM((K, D), jnp.float32),
             pltpu.VMEM_SHARED((NPAD, D), jnp.float32)]
            + ([pltpu.VMEM_SHARED((NPAD, D), jnp.float32)] if spmem_table
               else [])
            + [pltpu.SemaphoreType.DMA((4,)),
               pltpu.SemaphoreType.DMA((2,)),
               pltpu.SemaphoreType.DMA((2,))]
        ),
    )


_agg128 = _make_agg(DCOL, (0, 0), spmem_table=True, col_split=True)
_agg64 = _make_agg(DO, AGG64_SPLIT, spmem_table=True)


def _prescale_body(emb_ref, deg_ref, t1_ref, rso_ref, rsi_ref):
    deg = deg_ref[...]
    dego = deg[:, 0:8] + deg[:, 8:16]
    degi = deg[:, 16:24] + deg[:, 24:32]
    rso = lax.rsqrt(jnp.maximum(dego, 1.0))
    rsi = lax.rsqrt(jnp.maximum(degi, 1.0))
    rso_ref[...] = rso
    rsi_ref[...] = rsi
    t1_ref[...] = emb_ref[...] * rso[:, :1]


def _prescale(embp, deg):
    return pl.pallas_call(
        _prescale_body,
        grid=(GRID,),
        in_specs=[
            pl.BlockSpec((BLK, DF), lambda i: (i, 0)),
            pl.BlockSpec((BLK, DF), lambda i: (i, 0)),
        ],
        out_specs=[
            pl.BlockSpec((BLK, DF), lambda i: (i, 0)),
            pl.BlockSpec((BLK, DDEG), lambda i: (i, 0)),
            pl.BlockSpec((BLK, DDEG), lambda i: (i, 0)),
        ],
        out_shape=[
            jax.ShapeDtypeStruct((NPAD, DF), jnp.float32),
            jax.ShapeDtypeStruct((NPAD, DDEG), jnp.float32),
            jax.ShapeDtypeStruct((NPAD, DDEG), jnp.float32),
        ],
    )(embp, deg)


def _mid_body(agg_ref, rsi_ref, rso_ref, w1_ref, b1_ref, w2_ref, t2_ref):
    h = agg_ref[...] * rsi_ref[:, :1]
    h = jnp.dot(h, w1_ref[...], preferred_element_type=jnp.float32) + b1_ref[...]
    h = jnp.maximum(h, 0.0)
    h = h * rso_ref[:, :1]
    t2_ref[:, :DO] = jnp.dot(h, w2_ref[...], preferred_element_type=jnp.float32)


def _mid(agg1, rsi, rso, W1, b1, W2):
    return pl.pallas_call(
        _mid_body,
        grid=(GRID,),
        in_specs=[
            pl.BlockSpec((BLK, DF), lambda i: (i, 0)),
            pl.BlockSpec((BLK, DDEG), lambda i: (i, 0)),
            pl.BlockSpec((BLK, DDEG), lambda i: (i, 0)),
            pl.BlockSpec((DF, DH), lambda i: (0, 0)),
            pl.BlockSpec((1, DH), lambda i: (0, 0)),
            pl.BlockSpec((DH, DO), lambda i: (0, 0)),
        ],
        out_specs=pl.BlockSpec((BLK, DF), lambda i: (i, 0)),
        out_shape=jax.ShapeDtypeStruct((NPAD, DF), jnp.float32),
    )(agg1, rsi, rso, W1, b1, W2)


def _final_body(aggp_ref, rsi_ref, b2_ref, wf_ref, bf_ref, wb_ref, bb_ref, out_ref):
    aggp = aggp_ref[...]
    agg = aggp[:, :DO] + aggp[:, DO:]
    h2 = agg * rsi_ref[:, :1] + b2_ref[...]

    def lstm(w_ref, bias_ref):
        g = jnp.dot(h2, w_ref[...], preferred_element_type=jnp.float32) + bias_ref[...]
        gi = jax.nn.sigmoid(g[:, 0:32])
        gg = jnp.tanh(g[:, 64:96])
        go = jax.nn.sigmoid(g[:, 96:128])
        return go * jnp.tanh(gi * gg)

    out_ref[...] = jnp.concatenate([lstm(wf_ref, bf_ref), lstm(wb_ref, bb_ref)], axis=1)


def _final(aggp2, rsi, b2, wfT, bf, wbT, bb):
    blk = N // 10
    return pl.pallas_call(
        _final_body,
        grid=(10,),
        in_specs=[
            pl.BlockSpec((blk, DF), lambda i: (i, 0)),
            pl.BlockSpec((blk, DDEG), lambda i: (i, 0)),
            pl.BlockSpec((1, DO), lambda i: (0, 0)),
            pl.BlockSpec((DO, 4 * 32), lambda i: (0, 0)),
            pl.BlockSpec((1, 4 * 32), lambda i: (0, 0)),
            pl.BlockSpec((DO, 4 * 32), lambda i: (0, 0)),
            pl.BlockSpec((1, 4 * 32), lambda i: (0, 0)),
        ],
        out_specs=pl.BlockSpec((blk, DO), lambda i: (i, 0)),
        out_shape=jax.ShapeDtypeStruct((N, DO), jnp.float32),
    )(aggp2, rsi, b2, wfT, bf, wbT, bb)


def kernel(node_ids, edge_index, emb, W1, b1, W2, b2,
           Wih_f, Whh_f, bih_f, bhh_f, Wih_b, Whh_b, bih_b, bhh_b):
    f32 = jnp.float32
    src = edge_index[0]
    dst = edge_index[1]
    pad = jnp.full((EPAD - E,), N, jnp.int32)
    srcp = jnp.concatenate([src, pad]).reshape(TCH, K)
    dstp = jnp.concatenate([dst, pad]).reshape(TCH, K)
    embp = jnp.zeros((NPAD, DF), f32).at[:N].set(emb)

    edge3 = edge_index.reshape(2, TCH_RAW, K)
    deg = _deg_kernel(edge3[0], edge3[1])
    table1, rso, rsi = _prescale(embp, deg)
    aggp1 = _agg128(table1, srcp, dstp)
    table2 = _mid(aggp1, rsi, rso, W1, b1.reshape(1, DH), W2)
    aggp2 = _agg64(table2, srcp, dstp)
    bf = (bih_f + bhh_f).reshape(1, 4 * 32)
    bb = (bih_b + bhh_b).reshape(1, 4 * 32)
    return _final(aggp2, rsi, b2.reshape(1, DO), Wih_f.T, bf, Wih_b.T, bb)
